# baseline (speedup 1.0000x reference)
import functools

import jax
import jax.numpy as jnp
from jax import lax
from jax.experimental import pallas as pl
from jax.experimental.pallas import tpu as pltpu

B, C, H, W = 4, 192, 384, 384
E = 16
CH = C // 4
COLS = H * W
CB = 8
HC = C // 4
NC = HC // CB
NH = H // 8
NW = W // 128


def _accum(x4):
    acc = x4[0, :, 0:8, 0:128]
    for hg in range(NH):
        for wg in range(NW):
            if hg == 0 and wg == 0:
                continue
            acc = acc + x4[0, :, hg * 8:hg * 8 + 8, wg * 128:wg * 128 + 128]
    return acc.sum(axis=1)


def _body(xa_ref, xb_ref, xc_ref, xd_ref, w1_ref, b1_ref, w2_ref, b2_ref,
          o_ref, pacc_ref):
    b = pl.program_id(0)
    cb = pl.program_id(1)
    pacc_ref[b, pl.ds(cb * CB, CB), :] = _accum(xa_ref[...])
    pacc_ref[b, pl.ds(HC + cb * CB, CB), :] = _accum(xb_ref[...])
    pacc_ref[b, pl.ds(2 * HC + cb * CB, CB), :] = _accum(xc_ref[...])
    pacc_ref[b, pl.ds(3 * HC + cb * CB, CB), :] = _accum(xd_ref[...])

    @pl.when((b == B - 1) & (cb == NC - 1))
    def _():
        pooled = pacc_ref[...].sum(axis=2) * (1.0 / COLS)
        h = lax.dot_general(pooled, w1_ref[...],
                            (((1,), (1,)), ((), ())),
                            preferred_element_type=jnp.float32)
        h = jnp.maximum(h + b1_ref[...], 0.0)
        logits = lax.dot_general(h, w2_ref[...],
                                 (((1,), (1,)), ((), ())),
                                 preferred_element_type=jnp.float32)
        logits = logits + b2_ref[...]
        m = jnp.max(logits, axis=1, keepdims=True)
        e = jnp.exp(logits - m)
        o_ref[...] = e / jnp.sum(e, axis=1, keepdims=True)


@jax.jit
def kernel(x, w1, b1, w2, b2):
    return pl.pallas_call(
        _body,
        grid=(B, NC),
        in_specs=[
            pl.BlockSpec((1, CB, H, W), lambda b, c: (b, c, 0, 0)),
            pl.BlockSpec((1, CB, H, W), lambda b, c: (b, c + NC, 0, 0)),
            pl.BlockSpec((1, CB, H, W), lambda b, c: (b, c + 2 * NC, 0, 0)),
            pl.BlockSpec((1, CB, H, W), lambda b, c: (b, c + 3 * NC, 0, 0)),
            pl.BlockSpec((CH, C), lambda b, c: (0, 0)),
            pl.BlockSpec((1, CH), lambda b, c: (0, 0)),
            pl.BlockSpec((E, CH), lambda b, c: (0, 0)),
            pl.BlockSpec((1, E), lambda b, c: (0, 0)),
        ],
        out_specs=pl.BlockSpec((B, E), lambda b, c: (0, 0)),
        out_shape=jax.ShapeDtypeStruct((B, E), jnp.float32),
        scratch_shapes=[pltpu.VMEM((B, C, 128), jnp.float32)],
    )(x, x, x, x, w1, b1.reshape(1, CH), w2, b2.reshape(1, E))
